# baseline (speedup 1.0000x reference)
import functools

import jax
import jax.numpy as jnp
from jax import lax
from jax.experimental import pallas as pl
from jax.experimental.pallas import tpu as pltpu
from jax.experimental.pallas import tpu_sc as plsc

_B = 4096
_S = 200
_D = 32
_DO = _D + 1
_N = _B * _S
_NW = 32
_PER_W = _N // _NW
_CHUNK = 512
_NCHUNK = _PER_W // _CHUNK
_IDXW = 128
_GPC = _CHUNK // _IDXW
_L = 16


def _newton_sqrt(a):
    i = plsc.bitcast(a, jnp.int32)
    i = jnp.int32(0x5F3759DF) - lax.shift_right_logical(i, 1)
    y = plsc.bitcast(i, jnp.float32)
    for _ in range(3):
        y = y * (1.5 - 0.5 * a * y * y)
    return a * y


def _body(x_hbm, table_hbm, out_hbm, idx_v, rows_v, out_v, gsem):
    wid = lax.axis_index("s") * 2 + lax.axis_index("c")
    lane = lax.broadcasted_iota(jnp.int32, (_L,), 0)

    def chunk_body(c, carry):
        base = wid * _PER_W + c * _CHUNK
        pltpu.sync_copy(x_hbm.at[pl.ds(base, _CHUNK)], idx_v)
        descs = []
        for j in range(_GPC):
            descs.append(
                pltpu.async_copy(
                    table_hbm.at[idx_v.at[pl.ds(j * _IDXW, _IDXW)]],
                    rows_v.at[pl.ds(j * _IDXW, _IDXW)],
                    gsem,
                )
            )
        for d in descs:
            d.wait()

        def group_body(i, carry2):
            rvec = i * _L + lane
            acc = jnp.zeros((_L,), jnp.float32)
            for col in range(_D):
                cvec = jnp.full((_L,), col, jnp.int32)
                v = plsc.load_gather(rows_v, [rvec, cvec])
                acc = acc + v * v
                plsc.store_scatter(out_v, [rvec, cvec + 1], v)
            t = _newton_sqrt(acc + 1.0)
            plsc.store_scatter(out_v, [rvec, jnp.zeros((_L,), jnp.int32)], t)
            return carry2

        lax.fori_loop(0, _CHUNK // _L, group_body, 0, unroll=False)
        pltpu.sync_copy(out_v, out_hbm.at[pl.ds(base, _CHUNK)])
        return carry

    lax.fori_loop(0, _NCHUNK, chunk_body, 0, unroll=False)


@jax.jit
def _hembed_sc(xf, table):
    mesh = plsc.VectorSubcoreMesh(core_axis_name="c", subcore_axis_name="s")
    run = pl.kernel(
        _body,
        out_type=jax.ShapeDtypeStruct((_N, _DO), jnp.float32),
        mesh=mesh,
        scratch_types=[
            pltpu.VMEM((_CHUNK,), jnp.int32),
            pltpu.VMEM((_CHUNK, _D), jnp.float32),
            pltpu.VMEM((_CHUNK, _DO), jnp.float32),
            pltpu.SemaphoreType.DMA,
        ],
        compiler_params=pltpu.CompilerParams(
            needs_layout_passes=False, use_tc_tiling_on_sc=False
        ),
    )
    return run(xf, table)


def kernel(x, table):
    xf = x.reshape(_N).astype(jnp.int32)
    out = _hembed_sc(xf, table)
    return out.reshape(_B, _S, _DO)


# speedup vs baseline: 1.3355x; 1.3355x over previous
import jax
import jax.numpy as jnp
from jax import lax
from jax.experimental import pallas as pl
from jax.experimental.pallas import tpu as pltpu
from jax.experimental.pallas import tpu_sc as plsc

_B = 4096
_S = 200
_D = 32
_DO = _D + 1
_N = _B * _S
_NW = 32
_PER_W = _N // _NW
_C = 256
_NCH = _PER_W // _C
_IDXW = 128
_GPC = _C // _IDXW
_L = 16
_SLAB = 128
_NSLAB = 1000000 * _D // _SLAB


def _newton_sqrt(a):
    i = plsc.bitcast(a, jnp.int32)
    i = jnp.int32(0x5F3759DF) - lax.shift_right_logical(i, 1)
    y = plsc.bitcast(i, jnp.float32)
    for _ in range(3):
        y = y * (1.5 - 0.5 * a * y * y)
    return a * y


def _body(x_hbm, tslab_hbm, out_hbm,
          idx0, idx1, sidx0, sidx1, off0, off1,
          rows0, rows1, outv0, outv1,
          gsem0, gsem1, wsem0, wsem1):
    idx = (idx0, idx1)
    sidx = (sidx0, sidx1)
    off = (off0, off1)
    rows = (rows0, rows1)
    outv = (outv0, outv1)
    gsem = (gsem0, gsem1)
    wsem = (wsem0, wsem1)

    wid = lax.axis_index("s") * 2 + lax.axis_index("c")
    wbase = wid * _PER_W
    lane = lax.broadcasted_iota(jnp.int32, (_L,), 0)

    def stage(b, c):
        base = wbase + c * _C
        pltpu.sync_copy(x_hbm.at[pl.ds(base, _C)], idx[b])
        for k in range(_C // _L):
            v = idx[b][pl.ds(k * _L, _L)]
            sidx[b][pl.ds(k * _L, _L)] = lax.shift_right_logical(v, 2)
            off[b][pl.ds(k * _L, _L)] = (v & 3) * _D
        for j in range(_GPC):
            pltpu.async_copy(
                tslab_hbm.at[sidx[b].at[pl.ds(j * _IDXW, _IDXW)]],
                rows[b].at[pl.ds(j * _IDXW, _IDXW)],
                gsem[b],
            )

    def drain_gathers(b):
        for j in range(_GPC):
            pltpu.make_async_copy(
                tslab_hbm.at[sidx[b].at[pl.ds(j * _IDXW, _IDXW)]],
                rows[b].at[pl.ds(j * _IDXW, _IDXW)],
                gsem[b],
            ).wait()

    def fire_writes(b, c):
        base = wbase + c * _C
        for d in range(_DO):
            pltpu.async_copy(
                outv[b].at[pl.ds(d * _C, _C)],
                out_hbm.at[pl.ds(d * _N + base, _C)],
                wsem[b],
            )

    def drain_writes(b, c):
        base = wbase + c * _C
        for d in range(_DO):
            pltpu.make_async_copy(
                outv[b].at[pl.ds(d * _C, _C)],
                out_hbm.at[pl.ds(d * _N + base, _C)],
                wsem[b],
            ).wait()

    def compute(b):
        def group_body(g, carry):
            rvec = g * _L + lane
            o = off[b][pl.ds(g * _L, _L)]
            acc = jnp.zeros((_L,), jnp.float32)
            for col in range(_D):
                v = plsc.load_gather(rows[b], [rvec, o + col])
                acc = acc + v * v
                outv[b][pl.ds((col + 1) * _C + g * _L, _L)] = v
            outv[b][pl.ds(g * _L, _L)] = _newton_sqrt(acc + 1.0)
            return carry

        lax.fori_loop(0, _C // _L, group_body, 0, unroll=False)

    stage(0, 0)
    stage(1, 1)

    def pipe(cc, carry):
        c0 = 2 * cc
        c1 = c0 + 1
        for b, c in ((0, c0), (1, c1)):
            @pl.when(cc > 0)
            def _():
                drain_writes(b, c - 2)

            drain_gathers(b)
            compute(b)
            fire_writes(b, c)

            @pl.when(cc < _NCH // 2 - 1)
            def _():
                stage(b, c + 2)
        return carry

    lax.fori_loop(0, _NCH // 2, pipe, 0, unroll=False)
    drain_writes(0, _NCH - 2)
    drain_writes(1, _NCH - 1)


@jax.jit
def _hembed_sc(xf, tslab):
    mesh = plsc.VectorSubcoreMesh(core_axis_name="c", subcore_axis_name="s")
    run = pl.kernel(
        _body,
        out_type=jax.ShapeDtypeStruct((_DO * _N,), jnp.float32),
        mesh=mesh,
        scratch_types=[
            pltpu.VMEM((_C,), jnp.int32),
            pltpu.VMEM((_C,), jnp.int32),
            pltpu.VMEM((_C,), jnp.int32),
            pltpu.VMEM((_C,), jnp.int32),
            pltpu.VMEM((_C,), jnp.int32),
            pltpu.VMEM((_C,), jnp.int32),
            pltpu.VMEM((_C, _SLAB), jnp.float32),
            pltpu.VMEM((_C, _SLAB), jnp.float32),
            pltpu.VMEM((_DO * _C,), jnp.float32),
            pltpu.VMEM((_DO * _C,), jnp.float32),
            pltpu.SemaphoreType.DMA,
            pltpu.SemaphoreType.DMA,
            pltpu.SemaphoreType.DMA,
            pltpu.SemaphoreType.DMA,
        ],
        compiler_params=pltpu.CompilerParams(needs_layout_passes=False),
    )
    return run(xf, tslab)


def kernel(x, table):
    xs = x.astype(jnp.int32).T.reshape(_N)
    tslab = table.reshape(_NSLAB, _SLAB)
    out = _hembed_sc(xs, tslab)
    return out.reshape(_DO, _S, _B).transpose(2, 1, 0)


# speedup vs baseline: 1.4497x; 1.0855x over previous
import jax
import jax.numpy as jnp
from jax import lax
from jax.experimental import pallas as pl
from jax.experimental.pallas import tpu as pltpu
from jax.experimental.pallas import tpu_sc as plsc

_B = 4096
_S = 200
_D = 32
_DO = _D + 1
_N = _B * _S
_V = 1000000
_NW = 32
_PER_W = _N // _NW
_C = 128
_NCH = _PER_W // _C
_R = 4
_L = 16
_SLAB = 128
_TCV = 2048
_TCG = (_V + _TCV - 1) // _TCV
_NSLAB = _TCG * (_TCV // 4)


def _slab_body(t_ref, o_ref):
    t2 = t_ref[...].T
    q = _TCV // 4
    o_ref[...] = jnp.concatenate([t2[k * q:(k + 1) * q] for k in range(4)],
                                 axis=1)


def _make_slabs(table_t):
    return pl.pallas_call(
        _slab_body,
        grid=(_TCG,),
        in_specs=[pl.BlockSpec((_D, _TCV), lambda i: (0, i))],
        out_specs=pl.BlockSpec((_TCV // 4, _SLAB), lambda i: (i, 0)),
        out_shape=jax.ShapeDtypeStruct((_NSLAB, _SLAB), jnp.float32),
    )(table_t)


def _newton_sqrt(a):
    i = plsc.bitcast(a, jnp.int32)
    i = jnp.int32(0x5F3759DF) - lax.shift_right_logical(i, 1)
    y = plsc.bitcast(i, jnp.float32)
    for _ in range(3):
        y = y * (1.5 - 0.5 * a * y * y)
    return a * y


def _body(x_hbm, tslab_hbm, out_hbm,
          idx_all, sidx0, sidx1, sidx2, sidx3,
          rows0, rows1, rows2, rows3,
          outv0, outv1, outv2, outv3,
          gsem0, gsem1, gsem2, gsem3,
          wsem0, wsem1, wsem2, wsem3):
    sidx = (sidx0, sidx1, sidx2, sidx3)
    rows = (rows0, rows1, rows2, rows3)
    outv = (outv0, outv1, outv2, outv3)
    gsem = (gsem0, gsem1, gsem2, gsem3)
    wsem = (wsem0, wsem1, wsem2, wsem3)

    wid = lax.axis_index("s") * 2 + lax.axis_index("c")
    wbase = wid * _PER_W
    lane = lax.broadcasted_iota(jnp.int32, (_L,), 0)

    pltpu.sync_copy(x_hbm.at[pl.ds(wbase, _PER_W)], idx_all)

    def stage(r, c):
        for k in range(_C // _L):
            v = idx_all[pl.ds(c * _C + k * _L, _L)]
            sidx[r][pl.ds(k * _L, _L)] = (
                lax.shift_left(lax.shift_right_logical(v, 11), 9) | (v & 511)
            )
        pltpu.async_copy(tslab_hbm.at[sidx[r]], rows[r], gsem[r])

    def drain_gather(r):
        pltpu.make_async_copy(tslab_hbm.at[sidx[r]], rows[r], gsem[r]).wait()

    def fire_writes(r, c):
        base = wbase + c * _C
        for d in range(_DO):
            pltpu.async_copy(
                outv[r].at[pl.ds(d * _C, _C)],
                out_hbm.at[pl.ds(d * _N + base, _C)],
                wsem[r],
            )

    def drain_writes(r, c):
        base = wbase + c * _C
        for d in range(_DO):
            pltpu.make_async_copy(
                outv[r].at[pl.ds(d * _C, _C)],
                out_hbm.at[pl.ds(d * _N + base, _C)],
                wsem[r],
            ).wait()

    def compute(r, c):
        def group_body(g, carry):
            rvec = g * _L + lane
            v = idx_all[pl.ds(c * _C + g * _L, _L)]
            o = (lax.shift_right_logical(v, 9) & 3) * _D
            acc = jnp.zeros((_L,), jnp.float32)
            for col in range(_D):
                val = plsc.load_gather(rows[r], [rvec, o + col])
                acc = acc + val * val
                outv[r][pl.ds((col + 1) * _C + g * _L, _L)] = val
            outv[r][pl.ds(g * _L, _L)] = _newton_sqrt(acc + 1.0)
            return carry

        lax.fori_loop(0, _C // _L, group_body, 0, unroll=False)

    for r in range(_R):
        stage(r, r)

    def pipe(q, carry):
        for r in range(_R):
            c = q * _R + r

            @pl.when(q > 0)
            def _():
                drain_writes(r, c - _R)

            drain_gather(r)
            compute(r, c)
            fire_writes(r, c)

            @pl.when(q < _NCH // _R - 1)
            def _():
                stage(r, c + _R)
        return carry

    lax.fori_loop(0, _NCH // _R, pipe, 0, unroll=False)
    for r in range(_R):
        drain_writes(r, _NCH - _R + r)


@jax.jit
def _hembed_sc(xs, tslab):
    mesh = plsc.VectorSubcoreMesh(core_axis_name="c", subcore_axis_name="s")
    run = pl.kernel(
        _body,
        out_type=jax.ShapeDtypeStruct((_DO * _N,), jnp.float32),
        mesh=mesh,
        scratch_types=(
            [pltpu.VMEM((_PER_W,), jnp.int32)]
            + [pltpu.VMEM((_C,), jnp.int32) for _ in range(_R)]
            + [pltpu.VMEM((_C, _SLAB), jnp.float32) for _ in range(_R)]
            + [pltpu.VMEM((_DO * _C,), jnp.float32) for _ in range(_R)]
            + [pltpu.SemaphoreType.DMA for _ in range(2 * _R)]
        ),
        compiler_params=pltpu.CompilerParams(needs_layout_passes=False),
    )
    return run(xs, tslab)


def kernel(x, table):
    xs = x.astype(jnp.int32).T.reshape(_N)
    tslab = _make_slabs(table.T)
    out = _hembed_sc(xs, tslab)
    return out.reshape(_DO, _S, _B).transpose(2, 1, 0)


# speedup vs baseline: 3.3342x; 2.2999x over previous
import jax
import jax.numpy as jnp
from jax import lax
from jax.experimental import pallas as pl
from jax.experimental.pallas import tpu as pltpu
from jax.experimental.pallas import tpu_sc as plsc

_B = 4096
_S = 200
_D = 32
_DO = _D + 1
_N = _B * _S
_V = 1000000
_NW = 32
_PER_W = _N // _NW
_C = 128
_NCH = _PER_W // _C
_R = 4
_L = 16
_SLAB = 128
_TCV = 8192
_TCQ = _TCV // 4
_TCG = (_V + _TCV - 1) // _TCV
_NSLAB = _TCG * _TCQ
_QSH = _TCQ.bit_length() - 1
_TSH = _TCV.bit_length() - 1


def _slab_body(t_ref, o_ref):
    blk = t_ref[...]
    w = jnp.concatenate([blk[:, k * _TCQ:(k + 1) * _TCQ] for k in range(4)],
                        axis=0)
    o_ref[...] = w.T


def _make_slabs(table_t):
    return pl.pallas_call(
        _slab_body,
        grid=(_TCG,),
        in_specs=[pl.BlockSpec((_D, _TCV), lambda i: (0, i))],
        out_specs=pl.BlockSpec((_TCQ, _SLAB), lambda i: (i, 0)),
        out_shape=jax.ShapeDtypeStruct((_NSLAB, _SLAB), jnp.float32),
    )(table_t)


def _newton_sqrt(a):
    i = plsc.bitcast(a, jnp.int32)
    i = jnp.int32(0x5F3759DF) - lax.shift_right_logical(i, 1)
    y = plsc.bitcast(i, jnp.float32)
    for _ in range(2):
        y = y * (1.5 - 0.5 * a * y * y)
    return a * y


_SUP = 4
_SC_ = _C * _SUP


def _body(x_hbm, tslab_hbm, out_hbm,
          idx_all, sidx0, sidx1, sidx2, sidx3,
          rows0, rows1, rows2, rows3,
          outv0, outv1,
          gsem0, gsem1, gsem2, gsem3,
          wsem0, wsem1):
    sidx = (sidx0, sidx1, sidx2, sidx3)
    rows = (rows0, rows1, rows2, rows3)
    outv = (outv0, outv1)
    gsem = (gsem0, gsem1, gsem2, gsem3)
    wsem = (wsem0, wsem1)

    wid = lax.axis_index("s") * 2 + lax.axis_index("c")
    wbase = wid * _PER_W
    lane = lax.broadcasted_iota(jnp.int32, (_L,), 0)

    pltpu.sync_copy(x_hbm.at[pl.ds(wbase, _PER_W)], idx_all)

    def stage(r, c):
        for k in range(_C // _L):
            v = idx_all[pl.ds(c * _C + k * _L, _L)]
            sidx[r][pl.ds(k * _L, _L)] = (
                lax.shift_left(lax.shift_right_logical(v, _TSH), _QSH)
                | (v & (_TCQ - 1))
            )
        pltpu.async_copy(tslab_hbm.at[sidx[r]], rows[r], gsem[r])

    def drain_gather(r):
        pltpu.make_async_copy(tslab_hbm.at[sidx[r]], rows[r], gsem[r]).wait()

    def fire_writes(b, cs):
        base = wbase + cs * _SC_
        for d in range(_DO):
            pltpu.async_copy(
                outv[b].at[pl.ds(d * _SC_, _SC_)],
                out_hbm.at[pl.ds(d * _N + base, _SC_)],
                wsem[b],
            )

    def drain_writes(b, cs):
        del cs
        pltpu.make_async_copy(
            out_hbm.at[pl.ds(0, _DO * _SC_)], outv[b], wsem[b]
        ).wait()

    def compute(r, b, sub, c):
        @plsc.parallel_loop(0, _C // _L, 1, unroll=4)
        def _(g):
            rvec = g * _L + lane
            v = idx_all[pl.ds(c * _C + g * _L, _L)]
            o = (lax.shift_right_logical(v, _QSH) & 3) * _D
            accs = [jnp.zeros((_L,), jnp.float32) for _ in range(4)]
            for col in range(_D):
                val = plsc.load_gather(rows[r], [rvec, o + col])
                accs[col & 3] = accs[col & 3] + val * val
                outv[b][pl.ds((col + 1) * _SC_ + sub * _C + g * _L, _L)] = val
            acc = (accs[0] + accs[1]) + (accs[2] + accs[3])
            outv[b][pl.ds(sub * _C + g * _L, _L)] = _newton_sqrt(acc + 1.0)

    for r in range(_R):
        stage(r, r)

    def pipe(q, carry):
        for j in range(2 * _SUP):
            c = q * 2 * _SUP + j
            r = j % _R
            b = j // _SUP
            sub = j % _SUP

            if sub == 0:
                @pl.when(q > 0)
                def _():
                    drain_writes(b, 2 * q + j // _SUP - 2)

            drain_gather(r)
            compute(r, b, sub, c)

            if sub == _SUP - 1:
                fire_writes(b, 2 * q + j // _SUP)

            @pl.when(c + _R < _NCH)
            def _():
                stage(r, c + _R)
        return carry

    lax.fori_loop(0, _NCH // (2 * _SUP), pipe, 0, unroll=False)
    drain_writes(0, _NCH // _SUP - 2)
    drain_writes(1, _NCH // _SUP - 1)


@jax.jit
def _hembed_sc(xs, tslab):
    mesh = plsc.VectorSubcoreMesh(core_axis_name="c", subcore_axis_name="s")
    run = pl.kernel(
        _body,
        out_type=jax.ShapeDtypeStruct((_DO * _N,), jnp.float32),
        mesh=mesh,
        scratch_types=(
            [pltpu.VMEM((_PER_W,), jnp.int32)]
            + [pltpu.VMEM((_C,), jnp.int32) for _ in range(_R)]
            + [pltpu.VMEM((_C, _SLAB), jnp.float32) for _ in range(_R)]
            + [pltpu.VMEM((_DO * _SC_,), jnp.float32) for _ in range(2)]
            + [pltpu.SemaphoreType.DMA for _ in range(_R + 2)]
        ),
        compiler_params=pltpu.CompilerParams(needs_layout_passes=False),
    )
    return run(xs, tslab)


def kernel(x, table):
    xs = (x.astype(jnp.int32).T
          .reshape(_S // 8, 8, _B // 128, 128)
          .transpose(0, 2, 1, 3)
          .reshape(_N))
    tslab = _make_slabs(table.T)
    out = _hembed_sc(xs, tslab)
    return (out.reshape(_DO, _S // 8, _B // 128, 8, 128)
            .transpose(2, 4, 1, 3, 0)
            .reshape(_B, _S, _DO))
